# Initial kernel scaffold; baseline (speedup 1.0000x reference)
import jax
import jax.numpy as jnp
from jax.experimental import pallas as pl


def kernel(feat_0, feat_1, ei_00, ei_01, ei_10, ei_11, W1_00, W1_01, W1_10, W1_11, W2_00, W2_01, W2_10, W2_11):
    raise NotImplementedError("write your pallas kernel here")


# speedup vs baseline: 12.3776x; 12.3776x over previous
import functools

import jax
import jax.numpy as jnp
from jax import lax
from jax.experimental import pallas as pl
from jax.experimental.pallas import tpu as pltpu
from jax.experimental.pallas import tpu_sc as plsc

N = 10000
E = 320000
D_IN = 128
H1 = 64
H2 = 32

NC = 2
NS = 16
NW = NC * NS
E_PER_W = E // NW
CHUNK = 1000
NCHUNK = E_PER_W // CHUNK
N_PAD = 10240
ROWS_PER_TILE = N_PAD // NS


def _sc_agg_call(table, src_r, dst_r, zeros, d):
    mesh = plsc.VectorSubcoreMesh(core_axis_name="c", subcore_axis_name="s")

    @functools.partial(
        pl.kernel,
        mesh=mesh,
        compiler_params=pltpu.CompilerParams(use_tc_tiling_on_sc=False),
        out_type=jax.ShapeDtypeStruct((NC, N_PAD, d), jnp.float32),
        scratch_types=[
            pltpu.VMEM((CHUNK,), jnp.int32),
            pltpu.VMEM((CHUNK,), jnp.int32),
            pltpu.VMEM((CHUNK, d), jnp.float32),
            pltpu.VMEM_SHARED((N_PAD, d), jnp.float32),
            pltpu.SemaphoreType.DMA,
        ],
    )
    def k(table_hbm, src_hbm, dst_hbm, zeros_hbm, out_hbm,
          src_v, dst_v, rows_v, acc_sh, sem):
        c = lax.axis_index("c")
        s = lax.axis_index("s")
        w = c * NS + s
        row0 = s * ROWS_PER_TILE
        pltpu.sync_copy(zeros_hbm.at[pl.ds(row0, ROWS_PER_TILE)],
                        acc_sh.at[pl.ds(row0, ROWS_PER_TILE)])
        plsc.subcore_barrier()

        def body(i, carry):
            pltpu.sync_copy(src_hbm.at[w, i], src_v)
            pltpu.sync_copy(dst_hbm.at[w, i], dst_v)
            pltpu.async_copy(table_hbm.at[src_v], rows_v, sem).wait()
            pltpu.sync_copy(rows_v, acc_sh.at[dst_v], add=True)
            return carry

        lax.fori_loop(0, NCHUNK, body, 0)
        plsc.subcore_barrier()
        pltpu.sync_copy(acc_sh.at[pl.ds(row0, ROWS_PER_TILE)],
                        out_hbm.at[c, pl.ds(row0, ROWS_PER_TILE)])

    return k(table, src_r, dst_r, zeros)


def _edges(ei):
    ei = ei.astype(jnp.int32)
    src = ei[1].reshape(NW, NCHUNK, CHUNK)
    dst = ei[0].reshape(NW, NCHUNK, CHUNK)
    return src, dst


def _l2n(x):
    n = jnp.sqrt(jnp.maximum(jnp.sum(x * x, axis=1, keepdims=True), 1e-12))
    return x / n


_RB = 1000


def _t1_body(f0, f1, w00, w01, w10, w11, h00, h01, h10, h11):
    a = f0[...]
    b = f1[...]
    h00[...] = jnp.dot(a, w00[...], preferred_element_type=jnp.float32)
    h01[...] = jnp.dot(b, w01[...], preferred_element_type=jnp.float32)
    h10[...] = jnp.dot(a, w10[...], preferred_element_type=jnp.float32)
    h11[...] = jnp.dot(b, w11[...], preferred_element_type=jnp.float32)


def _t1(f0, f1, w00, w01, w10, w11):
    fs = pl.BlockSpec((_RB, D_IN), lambda i: (i, 0))
    ws = pl.BlockSpec((D_IN, H1), lambda i: (0, 0))
    os = pl.BlockSpec((_RB, H1), lambda i: (i, 0))
    sh = jax.ShapeDtypeStruct((N, H1), jnp.float32)
    return pl.pallas_call(
        _t1_body,
        grid=(N // _RB,),
        in_specs=[fs, fs, ws, ws, ws, ws],
        out_specs=[os, os, os, os],
        out_shape=[sh, sh, sh, sh],
    )(f0, f1, w00, w01, w10, w11)


def _t2_body(a00, a01, a10, a11, w00, w01, w10, w11, g00, g01, g10, g11):
    h0 = jax.nn.relu(_l2n(a00[0] + a00[1]) + _l2n(a01[0] + a01[1]))
    h1 = jax.nn.relu(_l2n(a10[0] + a10[1]) + _l2n(a11[0] + a11[1]))
    g00[...] = jnp.dot(h0, w00[...], preferred_element_type=jnp.float32)
    g01[...] = jnp.dot(h1, w01[...], preferred_element_type=jnp.float32)
    g10[...] = jnp.dot(h0, w10[...], preferred_element_type=jnp.float32)
    g11[...] = jnp.dot(h1, w11[...], preferred_element_type=jnp.float32)


def _t2(a00, a01, a10, a11, w00, w01, w10, w11):
    asp = pl.BlockSpec((NC, _RB, H1), lambda i: (0, i, 0))
    ws = pl.BlockSpec((H1, H2), lambda i: (0, 0))
    os = pl.BlockSpec((_RB, H2), lambda i: (i, 0))
    sh = jax.ShapeDtypeStruct((N, H2), jnp.float32)
    return pl.pallas_call(
        _t2_body,
        grid=(N // _RB,),
        in_specs=[asp, asp, asp, asp, ws, ws, ws, ws],
        out_specs=[os, os, os, os],
        out_shape=[sh, sh, sh, sh],
    )(a00, a01, a10, a11, w00, w01, w10, w11)


def _t3_body(b00, b01, b10, b11, e0, e1):
    e0[...] = _l2n(b00[0] + b00[1]) + _l2n(b01[0] + b01[1])
    e1[...] = _l2n(b10[0] + b10[1]) + _l2n(b11[0] + b11[1])


def _t3(b00, b01, b10, b11):
    bsp = pl.BlockSpec((NC, _RB, H2), lambda i: (0, i, 0))
    os = pl.BlockSpec((_RB, H2), lambda i: (i, 0))
    sh = jax.ShapeDtypeStruct((N, H2), jnp.float32)
    return pl.pallas_call(
        _t3_body,
        grid=(N // _RB,),
        in_specs=[bsp, bsp, bsp, bsp],
        out_specs=[os, os],
        out_shape=[sh, sh],
    )(b00, b01, b10, b11)


def kernel(feat_0, feat_1, ei_00, ei_01, ei_10, ei_11,
           W1_00, W1_01, W1_10, W1_11,
           W2_00, W2_01, W2_10, W2_11):
    s00, d00 = _edges(ei_00)
    s01, d01 = _edges(ei_01)
    s10, d10 = _edges(ei_10)
    s11, d11 = _edges(ei_11)
    z1 = jnp.zeros((N_PAD, H1), jnp.float32)
    z2 = jnp.zeros((N_PAD, H2), jnp.float32)

    h00, h01, h10, h11 = _t1(feat_0, feat_1, W1_00, W1_01, W1_10, W1_11)

    a00 = _sc_agg_call(h00, s00, d00, z1, H1)
    a01 = _sc_agg_call(h01, s01, d01, z1, H1)
    a10 = _sc_agg_call(h10, s10, d10, z1, H1)
    a11 = _sc_agg_call(h11, s11, d11, z1, H1)

    g00, g01, g10, g11 = _t2(a00, a01, a10, a11, W2_00, W2_01, W2_10, W2_11)

    b00 = _sc_agg_call(g00, s00, d00, z2, H2)
    b01 = _sc_agg_call(g01, s01, d01, z2, H2)
    b10 = _sc_agg_call(g10, s10, d10, z2, H2)
    b11 = _sc_agg_call(g11, s11, d11, z2, H2)

    e0, e1 = _t3(b00, b01, b10, b11)
    return jnp.concatenate([e0, e1], axis=0)
